# baseline (speedup 1.0000x reference)
import functools

import jax
import jax.numpy as jnp
from jax import lax
from jax.experimental import pallas as pl
from jax.experimental.pallas import tpu as pltpu
from jax.experimental.pallas import tpu_sc as plsc

NC = 2
NS = 16
NW = NC * NS
CH = 128


def _sc_mesh():
    return plsc.VectorSubcoreMesh(
        core_axis_name="c", subcore_axis_name="s", num_cores=NC, num_subcores=NS
    )


def _sc_gather(table, idx, cpw):
    e_pad = idx.shape[0]

    def body(table_hbm, idx_hbm, out_hbm, idx_v, rows_v, sem):
        wid = lax.axis_index("s") * NC + lax.axis_index("c")
        base = wid * (cpw * CH)

        def chunk(i, carry):
            off = base + i * CH
            pltpu.sync_copy(idx_hbm.at[pl.ds(off, CH)], idx_v)
            pltpu.async_copy(table_hbm.at[idx_v], rows_v, sem).wait()
            pltpu.sync_copy(rows_v, out_hbm.at[pl.ds(off, CH)])
            return carry

        lax.fori_loop(0, cpw, chunk, 0)

    return pl.kernel(
        body,
        out_type=jax.ShapeDtypeStruct((e_pad, 128), jnp.float32),
        mesh=_sc_mesh(),
        scratch_types=[
            pltpu.VMEM((CH,), jnp.int32),
            pltpu.VMEM((CH, 128), jnp.float32),
            pltpu.SemaphoreType.DMA,
        ],
    )(table, idx)


def _sc_scatter_add(rows, idx, zstripe, cpw):
    stripe = zstripe.shape[0]
    n_acc = stripe * NS

    def body(rows_hbm, idx_hbm, zero_hbm, out_hbm, idx_v, rows_v, acc_sh):
        c = lax.axis_index("c")
        s = lax.axis_index("s")
        wid = s * NC + c
        pltpu.sync_copy(zero_hbm, acc_sh.at[pl.ds(s * stripe, stripe)])
        plsc.subcore_barrier()

        base = wid * (cpw * CH)

        def chunk(i, carry):
            off = base + i * CH
            pltpu.sync_copy(idx_hbm.at[pl.ds(off, CH)], idx_v)
            pltpu.sync_copy(rows_hbm.at[pl.ds(off, CH)], rows_v)
            pltpu.sync_copy(rows_v, acc_sh.at[idx_v], add=True)
            return carry

        lax.fori_loop(0, cpw, chunk, 0)
        plsc.subcore_barrier()
        pltpu.sync_copy(acc_sh.at[pl.ds(s * stripe, stripe)],
                        out_hbm.at[c, pl.ds(s * stripe, stripe)])

    return pl.kernel(
        body,
        out_type=jax.ShapeDtypeStruct((NC, n_acc, 128), jnp.float32),
        mesh=_sc_mesh(),
        scratch_types=[
            pltpu.VMEM((CH,), jnp.int32),
            pltpu.VMEM((CH, 128), jnp.float32),
            pltpu.VMEM_SHARED((n_acc, 128), jnp.float32),
        ],
    )(rows, idx, zstripe)


def _node_proj(x, w0, b0):
    n = x.shape[0]
    nb = 1000

    def body(x_ref, w_ref, b_ref, o_ref):
        r = jnp.maximum(
            jnp.dot(x_ref[...], w_ref[...], preferred_element_type=jnp.float32)
            + b_ref[...],
            0.0,
        )
        o_ref[...] = jnp.concatenate([r, jnp.zeros_like(r)], axis=1)

    return pl.pallas_call(
        body,
        grid=(n // nb,),
        in_specs=[
            pl.BlockSpec((nb, 128), lambda i: (i, 0)),
            pl.BlockSpec((128, 64), lambda i: (0, 0)),
            pl.BlockSpec((1, 64), lambda i: (0, 0)),
        ],
        out_specs=pl.BlockSpec((nb, 128), lambda i: (i, 0)),
        out_shape=jax.ShapeDtypeStruct((n, 128), jnp.float32),
    )(x, w0, b0.reshape(1, 64))


def _edge_mlp1(ea, wn1, bn1):
    e_pad = ea.shape[0]
    eb = 2048

    def body(a_ref, w_ref, b_ref, o_ref):
        h = (
            jnp.dot(a_ref[...], w_ref[...], preferred_element_type=jnp.float32)
            + b_ref[...]
        )
        o_ref[...] = jnp.maximum(h, 0.0).astype(jnp.bfloat16)

    return pl.pallas_call(
        body,
        grid=(e_pad // eb,),
        in_specs=[
            pl.BlockSpec((eb, 4), lambda i: (i, 0)),
            pl.BlockSpec((4, 128), lambda i: (0, 0)),
            pl.BlockSpec((1, 128), lambda i: (0, 0)),
        ],
        out_specs=pl.BlockSpec((eb, 128), lambda i: (i, 0)),
        out_shape=jax.ShapeDtypeStruct((e_pad, 128), jnp.bfloat16),
    )(ea, wn1, bn1.reshape(1, 128))


def _msg(hrelu, u, wn2b, bn2r):
    e_pad = hrelu.shape[0]
    eb = 256

    def body(h_ref, u_ref, w_ref, b_ref, o_ref):
        ew = jnp.dot(h_ref[...], w_ref[...], preferred_element_type=jnp.float32)
        u = u_ref[:, :64]
        acc = jnp.dot(u, b_ref[...], preferred_element_type=jnp.float32)
        for i in range(64):
            acc = acc + u[:, i : i + 1] * ew[:, i * 64 : (i + 1) * 64]
        o_ref[...] = jnp.concatenate([acc, jnp.zeros_like(acc)], axis=1)

    return pl.pallas_call(
        body,
        grid=(e_pad // eb,),
        in_specs=[
            pl.BlockSpec((eb, 128), lambda i: (i, 0)),
            pl.BlockSpec((eb, 128), lambda i: (i, 0)),
            pl.BlockSpec((128, 4096), lambda i: (0, 0)),
            pl.BlockSpec((64, 64), lambda i: (0, 0)),
        ],
        out_specs=pl.BlockSpec((eb, 128), lambda i: (i, 0)),
        out_shape=jax.ShapeDtypeStruct((e_pad, 128), jnp.float32),
    )(hrelu, u, wn2b, bn2r)


def _gru(parts, inv_cnt, h, w_root, b_conv, wih_t, bih, whh_t, bhh):
    n = h.shape[0]
    nb = 1000

    def body(p_ref, ic_ref, h_ref, wr_ref, bc_ref, wi_ref, bi_ref, wh_ref,
             bh_ref, o_ref):
        hh = h_ref[:, :64]
        agg = (p_ref[0, :, :64] + p_ref[1, :, :64]) * ic_ref[...]
        m = jnp.maximum(
            agg
            + jnp.dot(hh, wr_ref[...], preferred_element_type=jnp.float32)
            + bc_ref[...],
            0.0,
        )
        gi = jnp.dot(m, wi_ref[...], preferred_element_type=jnp.float32) + bi_ref[...]
        gh = jnp.dot(hh, wh_ref[...], preferred_element_type=jnp.float32) + bh_ref[...]
        r = jax.nn.sigmoid(gi[:, :64] + gh[:, :64])
        z = jax.nn.sigmoid(gi[:, 64:128] + gh[:, 64:128])
        nn = jnp.tanh(gi[:, 128:] + r * gh[:, 128:])
        hnew = (1.0 - z) * nn + z * hh
        o_ref[...] = jnp.concatenate([hnew, jnp.zeros_like(hnew)], axis=1)

    return pl.pallas_call(
        body,
        grid=(n // nb,),
        in_specs=[
            pl.BlockSpec((2, nb, 128), lambda i: (0, i, 0)),
            pl.BlockSpec((nb, 64), lambda i: (i, 0)),
            pl.BlockSpec((nb, 128), lambda i: (i, 0)),
            pl.BlockSpec((64, 64), lambda i: (0, 0)),
            pl.BlockSpec((1, 64), lambda i: (0, 0)),
            pl.BlockSpec((64, 192), lambda i: (0, 0)),
            pl.BlockSpec((1, 192), lambda i: (0, 0)),
            pl.BlockSpec((64, 192), lambda i: (0, 0)),
            pl.BlockSpec((1, 192), lambda i: (0, 0)),
        ],
        out_specs=pl.BlockSpec((nb, 128), lambda i: (i, 0)),
        out_shape=jax.ShapeDtypeStruct((n, 128), jnp.float32),
    )(parts, inv_cnt, h, w_root, b_conv, wih_t, bih, whh_t, bhh)


def _set2set(h, batch2d, wih_t, bih, whh_t, bhh, w1, b1, w2, b2):
    n = h.shape[0]
    nb_graphs = 64

    def body(h_ref, b_ref, wi_ref, bi_ref, wh_ref, bh_ref, w1_ref, b1_ref,
             w2_ref, b2_ref, o_ref):
        hv = h_ref[:, :64]
        onehot = (
            b_ref[...]
            == lax.broadcasted_iota(jnp.int32, (1, nb_graphs), 1)
        ).astype(jnp.float32)
        neg = (1.0 - onehot) * (-1e30)
        q_star = jnp.zeros((nb_graphs, 128), jnp.float32)
        hl = jnp.zeros((nb_graphs, 64), jnp.float32)
        cl = jnp.zeros((nb_graphs, 64), jnp.float32)
        for _ in range(3):
            gates = (
                jnp.dot(q_star, wi_ref[...], preferred_element_type=jnp.float32)
                + bi_ref[...]
                + jnp.dot(hl, wh_ref[...], preferred_element_type=jnp.float32)
                + bh_ref[...]
            )
            ii = jax.nn.sigmoid(gates[:, :64])
            ff = jax.nn.sigmoid(gates[:, 64:128])
            gg = jnp.tanh(gates[:, 128:192])
            oo = jax.nn.sigmoid(gates[:, 192:])
            cl = ff * cl + ii * gg
            hl = oo * jnp.tanh(cl)
            q = hl
            e_all = lax.dot_general(
                hv, q, (((1,), (1,)), ((), ())),
                preferred_element_type=jnp.float32,
            )
            e = jnp.sum(e_all * onehot, axis=1, keepdims=True)
            emax = jnp.max(e_all * onehot + neg, axis=0, keepdims=True)
            emax_row = jnp.sum(emax * onehot, axis=1, keepdims=True)
            ex = jnp.exp(e - emax_row)
            den = jnp.sum(ex * onehot, axis=0, keepdims=True)
            den_row = jnp.sum(den * onehot, axis=1, keepdims=True)
            a = ex / (den_row + 1e-16)
            r_ = lax.dot_general(
                onehot, a * hv, (((0,), (0,)), ((), ())),
                preferred_element_type=jnp.float32,
            )
            q_star = jnp.concatenate([q, r_], axis=1)
        z1 = jnp.maximum(
            jnp.dot(q_star, w1_ref[...], preferred_element_type=jnp.float32)
            + b1_ref[...],
            0.0,
        )
        o_ref[...] = (
            jnp.dot(z1, w2_ref[...], preferred_element_type=jnp.float32)
            + b2_ref[...]
        )

    return pl.pallas_call(
        body,
        out_shape=jax.ShapeDtypeStruct((nb_graphs, 19), jnp.float32),
    )(h, batch2d, wih_t, bih, whh_t, bhh, w1, b1, w2, b2)


def kernel(x, edge_index, edge_attr, batch, W0, b0, Wn1, bn1, Wn2, bn2,
           W_root, b_conv, gru_Wih, gru_Whh, gru_bih, gru_bhh, lstm_Wih,
           lstm_Whh, lstm_bih, lstm_bhh, W1, b1, W2, b2):
    n = x.shape[0]
    e = edge_index.shape[1]
    cpw = -(-e // (NW * CH))
    e_pad = NW * CH * cpw
    n_acc = ((n + 1 + 127) // 128) * 128

    src = edge_index[0]
    dst = edge_index[1]
    src_p = jnp.concatenate([src, jnp.zeros((e_pad - e,), jnp.int32)])
    dst_p = jnp.concatenate([dst, jnp.full((e_pad - e,), n, jnp.int32)])
    ea_p = jnp.concatenate(
        [edge_attr, jnp.zeros((e_pad - e, 4), jnp.float32)], axis=0
    )
    zstripe = jnp.zeros((n_acc // NS, 128), jnp.float32)

    h = _node_proj(x, W0, b0)
    hrelu = _edge_mlp1(ea_p, Wn1, bn1)
    wn2b = Wn2.astype(jnp.bfloat16)
    bn2r = bn2.reshape(64, 64)

    cnt_parts = _sc_scatter_add(jnp.ones((e_pad, 128), jnp.float32), dst_p,
                                zstripe, cpw)
    cnt = cnt_parts[0, :n, 0] + cnt_parts[1, :n, 0]
    inv_cnt = jnp.broadcast_to(
        (1.0 / jnp.maximum(cnt, 1.0))[:, None], (n, 64)
    )

    wih_t = gru_Wih.T
    whh_t = gru_Whh.T
    for _ in range(3):
        u = _sc_gather(h, src_p, cpw)
        msg = _msg(hrelu, u, wn2b, bn2r)
        parts = _sc_scatter_add(msg, dst_p, zstripe, cpw)
        h = _gru(parts, inv_cnt, h, W_root, b_conv.reshape(1, 64), wih_t,
                 gru_bih.reshape(1, 192), whh_t, gru_bhh.reshape(1, 192))

    out19 = _set2set(h, batch.reshape(n, 1), lstm_Wih.T,
                     lstm_bih.reshape(1, 256), lstm_Whh.T,
                     lstm_bhh.reshape(1, 256), W1, b1.reshape(1, 64), W2,
                     b2.reshape(1, 19))
    return out19.reshape(-1)


# speedup vs baseline: 3.0893x; 3.0893x over previous
import functools

import jax
import jax.numpy as jnp
from jax import lax
from jax.experimental import pallas as pl
from jax.experimental.pallas import tpu as pltpu
from jax.experimental.pallas import tpu_sc as plsc

NC = 2
NS = 16
NW = NC * NS
CH = 128


def _sc_mesh():
    return plsc.VectorSubcoreMesh(
        core_axis_name="c", subcore_axis_name="s", num_cores=NC, num_subcores=NS
    )


def _sc_gather(table, idx, cpw):
    e_pad = idx.shape[0]

    def body(table_hbm, idx_hbm, out_hbm, idx_v, rows_v, sem):
        wid = lax.axis_index("s") * NC + lax.axis_index("c")
        base = wid * (cpw * CH)

        def chunk(i, carry):
            off = base + i * CH
            pltpu.sync_copy(idx_hbm.at[pl.ds(off, CH)], idx_v)
            pltpu.async_copy(table_hbm.at[idx_v], rows_v, sem).wait()
            pltpu.sync_copy(rows_v, out_hbm.at[pl.ds(off, CH)])
            return carry

        lax.fori_loop(0, cpw, chunk, 0)

    return pl.kernel(
        body,
        out_type=jax.ShapeDtypeStruct((e_pad, 128), jnp.float32),
        mesh=_sc_mesh(),
        scratch_types=[
            pltpu.VMEM((CH,), jnp.int32),
            pltpu.VMEM((CH, 128), jnp.float32),
            pltpu.SemaphoreType.DMA,
        ],
    )(table, idx)


def _sc_scatter_add(rows, idx, zstripe, cpw):
    stripe = zstripe.shape[0]
    n_acc = stripe * NS

    def body(rows_hbm, idx_hbm, zero_hbm, out_hbm, idx_v, rows_v, acc_sh):
        c = lax.axis_index("c")
        s = lax.axis_index("s")
        wid = s * NC + c
        pltpu.sync_copy(zero_hbm, acc_sh.at[pl.ds(s * stripe, stripe)])
        plsc.subcore_barrier()

        base = wid * (cpw * CH)

        def chunk(i, carry):
            off = base + i * CH
            pltpu.sync_copy(idx_hbm.at[pl.ds(off, CH)], idx_v)
            pltpu.sync_copy(rows_hbm.at[pl.ds(off, CH)], rows_v)
            pltpu.sync_copy(rows_v, acc_sh.at[idx_v], add=True)
            return carry

        lax.fori_loop(0, cpw, chunk, 0)
        plsc.subcore_barrier()
        pltpu.sync_copy(acc_sh.at[pl.ds(s * stripe, stripe)],
                        out_hbm.at[c, pl.ds(s * stripe, stripe)])

    return pl.kernel(
        body,
        out_type=jax.ShapeDtypeStruct((NC, n_acc, 128), jnp.float32),
        mesh=_sc_mesh(),
        scratch_types=[
            pltpu.VMEM((CH,), jnp.int32),
            pltpu.VMEM((CH, 128), jnp.float32),
            pltpu.VMEM_SHARED((n_acc, 128), jnp.float32),
        ],
    )(rows, idx, zstripe)


def _node_proj(x, w0, b0):
    n = x.shape[0]
    nb = 1000

    def body(x_ref, w_ref, b_ref, o_ref):
        r = jnp.maximum(
            jnp.dot(x_ref[...], w_ref[...], preferred_element_type=jnp.float32)
            + b_ref[...],
            0.0,
        )
        o_ref[...] = jnp.concatenate([r, jnp.zeros_like(r)], axis=1)

    return pl.pallas_call(
        body,
        grid=(n // nb,),
        in_specs=[
            pl.BlockSpec((nb, 128), lambda i: (i, 0)),
            pl.BlockSpec((128, 64), lambda i: (0, 0)),
            pl.BlockSpec((1, 64), lambda i: (0, 0)),
        ],
        out_specs=pl.BlockSpec((nb, 128), lambda i: (i, 0)),
        out_shape=jax.ShapeDtypeStruct((n, 128), jnp.float32),
    )(x, w0, b0.reshape(1, 64))


def _edge_mlp1t(ea_t, wn1_t, bn1_c):
    e_pad = ea_t.shape[1]
    eb = 2048

    def body(a_ref, w_ref, b_ref, o_ref):
        h = (
            jnp.dot(w_ref[...], a_ref[...], preferred_element_type=jnp.float32)
            + b_ref[...]
        )
        o_ref[...] = jnp.maximum(h, 0.0).astype(jnp.bfloat16)

    return pl.pallas_call(
        body,
        grid=(e_pad // eb,),
        in_specs=[
            pl.BlockSpec((4, eb), lambda i: (0, i)),
            pl.BlockSpec((128, 4), lambda i: (0, 0)),
            pl.BlockSpec((128, 1), lambda i: (0, 0)),
        ],
        out_specs=pl.BlockSpec((128, eb), lambda i: (0, i)),
        out_shape=jax.ShapeDtypeStruct((128, e_pad), jnp.bfloat16),
    )(ea_t, wn1_t, bn1_c)


def _msg(hrelu_t, u, wn2_t, bn2r_t):
    e_pad = hrelu_t.shape[1]
    eb = 256

    def body(h_ref, u_ref, w_ref, b_ref, o_ref):
        ewt = jnp.dot(w_ref[...], h_ref[...], preferred_element_type=jnp.float32)
        ut = jnp.transpose(u_ref[:, :64])
        acc = jnp.dot(b_ref[...], ut, preferred_element_type=jnp.float32)
        ew3 = ewt.reshape(64, 64, eb)
        for i in range(64):
            acc = acc + ut[i : i + 1, :] * ew3[i]
        o_ref[...] = jnp.concatenate(
            [jnp.transpose(acc), jnp.zeros((eb, 64), jnp.float32)], axis=1
        )

    return pl.pallas_call(
        body,
        grid=(e_pad // eb,),
        in_specs=[
            pl.BlockSpec((128, eb), lambda i: (0, i)),
            pl.BlockSpec((eb, 128), lambda i: (i, 0)),
            pl.BlockSpec((4096, 128), lambda i: (0, 0)),
            pl.BlockSpec((64, 64), lambda i: (0, 0)),
        ],
        out_specs=pl.BlockSpec((eb, 128), lambda i: (i, 0)),
        out_shape=jax.ShapeDtypeStruct((e_pad, 128), jnp.float32),
    )(hrelu_t, u, wn2_t, bn2r_t)


def _gru(parts, inv_cnt, h, w_root, b_conv, wih_t, bih, whh_t, bhh):
    n = h.shape[0]
    nb = 1000

    def body(p_ref, ic_ref, h_ref, wr_ref, bc_ref, wi_ref, bi_ref, wh_ref,
             bh_ref, o_ref):
        hh = h_ref[:, :64]
        agg = (p_ref[0, :, :64] + p_ref[1, :, :64]) * ic_ref[...]
        m = jnp.maximum(
            agg
            + jnp.dot(hh, wr_ref[...], preferred_element_type=jnp.float32)
            + bc_ref[...],
            0.0,
        )
        gi = jnp.dot(m, wi_ref[...], preferred_element_type=jnp.float32) + bi_ref[...]
        gh = jnp.dot(hh, wh_ref[...], preferred_element_type=jnp.float32) + bh_ref[...]
        r = jax.nn.sigmoid(gi[:, :64] + gh[:, :64])
        z = jax.nn.sigmoid(gi[:, 64:128] + gh[:, 64:128])
        nn = jnp.tanh(gi[:, 128:] + r * gh[:, 128:])
        hnew = (1.0 - z) * nn + z * hh
        o_ref[...] = jnp.concatenate([hnew, jnp.zeros_like(hnew)], axis=1)

    return pl.pallas_call(
        body,
        grid=(n // nb,),
        in_specs=[
            pl.BlockSpec((2, nb, 128), lambda i: (0, i, 0)),
            pl.BlockSpec((nb, 64), lambda i: (i, 0)),
            pl.BlockSpec((nb, 128), lambda i: (i, 0)),
            pl.BlockSpec((64, 64), lambda i: (0, 0)),
            pl.BlockSpec((1, 64), lambda i: (0, 0)),
            pl.BlockSpec((64, 192), lambda i: (0, 0)),
            pl.BlockSpec((1, 192), lambda i: (0, 0)),
            pl.BlockSpec((64, 192), lambda i: (0, 0)),
            pl.BlockSpec((1, 192), lambda i: (0, 0)),
        ],
        out_specs=pl.BlockSpec((nb, 128), lambda i: (i, 0)),
        out_shape=jax.ShapeDtypeStruct((n, 128), jnp.float32),
    )(parts, inv_cnt, h, w_root, b_conv, wih_t, bih, whh_t, bhh)


def _set2set(h, batch2d, wih_t, bih, whh_t, bhh, w1, b1, w2, b2):
    n = h.shape[0]
    nb_graphs = 64

    def body(h_ref, b_ref, wi_ref, bi_ref, wh_ref, bh_ref, w1_ref, b1_ref,
             w2_ref, b2_ref, o_ref):
        hv = h_ref[:, :64]
        onehot = (
            b_ref[...]
            == lax.broadcasted_iota(jnp.int32, (1, nb_graphs), 1)
        ).astype(jnp.float32)
        neg = (1.0 - onehot) * (-1e30)
        q_star = jnp.zeros((nb_graphs, 128), jnp.float32)
        hl = jnp.zeros((nb_graphs, 64), jnp.float32)
        cl = jnp.zeros((nb_graphs, 64), jnp.float32)
        for _ in range(3):
            gates = (
                jnp.dot(q_star, wi_ref[...], preferred_element_type=jnp.float32)
                + bi_ref[...]
                + jnp.dot(hl, wh_ref[...], preferred_element_type=jnp.float32)
                + bh_ref[...]
            )
            ii = jax.nn.sigmoid(gates[:, :64])
            ff = jax.nn.sigmoid(gates[:, 64:128])
            gg = jnp.tanh(gates[:, 128:192])
            oo = jax.nn.sigmoid(gates[:, 192:])
            cl = ff * cl + ii * gg
            hl = oo * jnp.tanh(cl)
            q = hl
            e_all = lax.dot_general(
                hv, q, (((1,), (1,)), ((), ())),
                preferred_element_type=jnp.float32,
            )
            e = jnp.sum(e_all * onehot, axis=1, keepdims=True)
            emax = jnp.max(e_all * onehot + neg, axis=0, keepdims=True)
            emax_row = jnp.sum(emax * onehot, axis=1, keepdims=True)
            ex = jnp.exp(e - emax_row)
            den = jnp.sum(ex * onehot, axis=0, keepdims=True)
            den_row = jnp.sum(den * onehot, axis=1, keepdims=True)
            a = ex / (den_row + 1e-16)
            r_ = lax.dot_general(
                onehot, a * hv, (((0,), (0,)), ((), ())),
                preferred_element_type=jnp.float32,
            )
            q_star = jnp.concatenate([q, r_], axis=1)
        z1 = jnp.maximum(
            jnp.dot(q_star, w1_ref[...], preferred_element_type=jnp.float32)
            + b1_ref[...],
            0.0,
        )
        o_ref[...] = (
            jnp.dot(z1, w2_ref[...], preferred_element_type=jnp.float32)
            + b2_ref[...]
        )

    return pl.pallas_call(
        body,
        out_shape=jax.ShapeDtypeStruct((nb_graphs, 19), jnp.float32),
    )(h, batch2d, wih_t, bih, whh_t, bhh, w1, b1, w2, b2)


def kernel(x, edge_index, edge_attr, batch, W0, b0, Wn1, bn1, Wn2, bn2,
           W_root, b_conv, gru_Wih, gru_Whh, gru_bih, gru_bhh, lstm_Wih,
           lstm_Whh, lstm_bih, lstm_bhh, W1, b1, W2, b2):
    n = x.shape[0]
    e = edge_index.shape[1]
    cpw = -(-e // (NW * CH))
    e_pad = NW * CH * cpw
    n_acc = ((n + 1 + 127) // 128) * 128

    src = edge_index[0]
    dst = edge_index[1]
    src_p = jnp.concatenate([src, jnp.zeros((e_pad - e,), jnp.int32)])
    dst_p = jnp.concatenate([dst, jnp.full((e_pad - e,), n, jnp.int32)])
    ea_t = jnp.concatenate(
        [edge_attr, jnp.zeros((e_pad - e, 4), jnp.float32)], axis=0
    ).T
    zstripe = jnp.zeros((n_acc // NS, 128), jnp.float32)

    h = _node_proj(x, W0, b0)
    hrelu_t = _edge_mlp1t(ea_t, Wn1.T, bn1.reshape(128, 1))
    wn2_t = Wn2.T.astype(jnp.bfloat16)
    bn2r_t = bn2.reshape(64, 64).T

    cnt_parts = _sc_scatter_add(jnp.ones((e_pad, 128), jnp.float32), dst_p,
                                zstripe, cpw)
    cnt = cnt_parts[0, :n, 0] + cnt_parts[1, :n, 0]
    inv_cnt = jnp.broadcast_to(
        (1.0 / jnp.maximum(cnt, 1.0))[:, None], (n, 64)
    )

    wih_t = gru_Wih.T
    whh_t = gru_Whh.T
    for _ in range(3):
        u = _sc_gather(h, src_p, cpw)
        msg = _msg(hrelu_t, u, wn2_t, bn2r_t)
        parts = _sc_scatter_add(msg, dst_p, zstripe, cpw)
        h = _gru(parts, inv_cnt, h, W_root, b_conv.reshape(1, 64), wih_t,
                 gru_bih.reshape(1, 192), whh_t, gru_bhh.reshape(1, 192))

    out19 = _set2set(h, batch.reshape(n, 1), lstm_Wih.T,
                     lstm_bih.reshape(1, 256), lstm_Whh.T,
                     lstm_bhh.reshape(1, 256), W1, b1.reshape(1, 64), W2,
                     b2.reshape(1, 19))
    return out19.reshape(-1)
